# baseline (speedup 1.0000x reference)
import functools

import jax
import jax.numpy as jnp
from jax import lax
from jax.experimental import pallas as pl
from jax.experimental.pallas import tpu as pltpu
from jax.experimental.pallas import tpu_sc as plsc

LANES = 16
IDX_PER_DMA = 512
NBUF = 2
NIDX = 4


@functools.lru_cache(maxsize=None)
def _build(n_tok: int, q: int, f: int):
    info = plsc.get_sparse_core_info()
    nc, ns = info.num_cores, info.num_subcores
    nw = nc * ns

    assert f % LANES == 0
    assert n_tok % nw == 0
    tok_w = n_tok // nw
    t_chunk = 64
    while tok_w % (t_chunk * NIDX):
        t_chunk //= 2
    n_chunks = tok_w // t_chunk
    assert n_chunks >= NIDX
    rows_chunk = t_chunk * q
    assert rows_chunk % IDX_PER_DMA == 0
    n_sub = rows_chunk // IDX_PER_DMA
    f_v = f // LANES

    mesh = plsc.VectorSubcoreMesh(core_axis_name="c", subcore_axis_name="s")

    @functools.partial(
        pl.kernel,
        out_type=jax.ShapeDtypeStruct((n_tok, f), jnp.float32),
        mesh=mesh,
        compiler_params=pltpu.CompilerParams(use_tc_tiling_on_sc=False),
        scratch_types=[
            pltpu.VMEM((NIDX, rows_chunk), jnp.int32),
            pltpu.VMEM((NBUF, rows_chunk, f), jnp.float32),
            pltpu.VMEM((NBUF, t_chunk, f), jnp.float32),
            pltpu.VMEM((LANES,), jnp.int32),
            pltpu.SemaphoreType.DMA,
            pltpu.SemaphoreType.DMA,
            pltpu.SemaphoreType.DMA,
            pltpu.SemaphoreType.DMA,
            pltpu.SemaphoreType.DMA,
            pltpu.SemaphoreType.DMA,
            pltpu.SemaphoreType.DMA,
            pltpu.SemaphoreType.DMA,
        ],
    )
    def emb(xf_hbm, table_hbm, off_hbm, out_hbm, idx_v, rows_v, out_v, off_v,
            gsem0, gsem1, isem0, isem1, isem2, isem3, osem0, osem1):
        gsems = (gsem0, gsem1)
        isems = (isem0, isem1, isem2, isem3)
        osems = (osem0, osem1)
        wid = lax.axis_index("s") * nc + lax.axis_index("c")
        base_t = wid * tok_w

        pltpu.sync_copy(off_hbm, off_v)
        offv = off_v[...]

        def idx_desc(si, g):
            return pltpu.make_async_copy(
                xf_hbm.at[pl.ds((base_t + g * t_chunk) * q, rows_chunk)],
                idx_v.at[si], isems[si])

        def gather_descs(b, si):
            return [
                pltpu.make_async_copy(
                    table_hbm.at[idx_v.at[si, pl.ds(j * IDX_PER_DMA, IDX_PER_DMA)]],
                    rows_v.at[b, pl.ds(j * IDX_PER_DMA, IDX_PER_DMA)],
                    gsems[b],
                )
                for j in range(n_sub)
            ]

        def out_desc(b, g):
            return pltpu.make_async_copy(
                out_v.at[b], out_hbm.at[pl.ds(base_t + g * t_chunk, t_chunk)],
                osems[b])

        def stage(b, si, g):
            idx_desc(si, g).wait()

            @plsc.parallel_loop(0, rows_chunk, LANES, unroll=2)
            def off_body(s):
                idx_v[si, pl.ds(s, LANES)] = idx_v[si, pl.ds(s, LANES)] + offv

            for h in gather_descs(b, si):
                h.start()

        def compute(b, g):
            @pl.when(g >= NBUF)
            def _():
                out_desc(b, g).wait()

            @plsc.parallel_loop(0, t_chunk, 1, unroll=2)
            def tok_body(t):
                rbase = t * q
                for cc in range(f_v):
                    sl = pl.ds(cc * LANES, LANES)
                    vals = [rows_v[b, rbase + qq, sl] for qq in range(q)]
                    while len(vals) > 1:
                        vals = [vals[i] + vals[i + 1] for i in range(0, len(vals) - 1, 2)] + (
                            [vals[-1]] if len(vals) % 2 else [])
                    out_v[b, t, sl] = vals[0]

            out_desc(b, g).start()

        for g0 in range(NIDX):
            idx_desc(g0, g0).start()
        for g0 in range(NBUF):
            stage(g0 % NBUF, g0 % NIDX, g0)

        def loop_body(i, c):
            gg = i * NIDX
            for k in range(NIDX):
                g = gg + k
                b = k % NBUF
                si = k
                sj = (k + NBUF) % NIDX

                for h in gather_descs(b, si):
                    h.wait()

                @pl.when(g + NIDX < n_chunks)
                def _():
                    idx_desc(si, g + NIDX).start()

                compute(b, g)

                @pl.when(g + NBUF < n_chunks)
                def _():
                    stage(b, sj, g + NBUF)

            return c

        lax.fori_loop(0, n_chunks // NIDX, loop_body, 0)

        for g0 in range(NBUF):
            out_desc(g0 % NBUF, n_chunks - NBUF + g0).wait()

    return emb


def kernel(x, table, sum_over_quantizers):
    b, s, q = x.shape
    v, f = table.shape
    seg = v // q
    n_tok = b * s

    flag = jnp.asarray(sum_over_quantizers).astype(jnp.int32)
    off16 = (jnp.arange(LANES, dtype=jnp.int32) % q) * jnp.int32(seg) * flag

    xf = x.reshape(n_tok * q)
    out = _build(n_tok, q, f)(xf, table, off16)
    return out.reshape(b, s, f)


# speedup vs baseline: 1.0015x; 1.0015x over previous
import functools

import jax
import jax.numpy as jnp
from jax import lax
from jax.experimental import pallas as pl
from jax.experimental.pallas import tpu as pltpu
from jax.experimental.pallas import tpu_sc as plsc

LANES = 16
IDX_PER_DMA = 512
NBUF = 2
NIDX = 4


@functools.lru_cache(maxsize=None)
def _build(n_tok: int, q: int, f: int):
    info = plsc.get_sparse_core_info()
    nc, ns = info.num_cores, info.num_subcores
    nw = nc * ns

    assert f % LANES == 0
    assert n_tok % nw == 0
    tok_w = n_tok // nw
    t_chunk = 64
    while tok_w % (t_chunk * NIDX):
        t_chunk //= 2
    n_chunks = tok_w // t_chunk
    assert n_chunks >= NIDX
    rows_chunk = t_chunk * q
    assert rows_chunk % IDX_PER_DMA == 0
    n_sub = rows_chunk // IDX_PER_DMA
    f_v = f // LANES

    mesh = plsc.VectorSubcoreMesh(core_axis_name="c", subcore_axis_name="s")

    @functools.partial(
        pl.kernel,
        out_type=jax.ShapeDtypeStruct((n_tok, f), jnp.float32),
        mesh=mesh,
        compiler_params=pltpu.CompilerParams(use_tc_tiling_on_sc=False, needs_layout_passes=False),
        scratch_types=[
            pltpu.VMEM((NIDX, rows_chunk), jnp.int32),
            pltpu.VMEM((NBUF, rows_chunk, f), jnp.float32),
            pltpu.VMEM((NBUF, t_chunk, f), jnp.float32),
            pltpu.VMEM((LANES,), jnp.int32),
            pltpu.SemaphoreType.DMA,
            pltpu.SemaphoreType.DMA,
            pltpu.SemaphoreType.DMA,
            pltpu.SemaphoreType.DMA,
            pltpu.SemaphoreType.DMA,
            pltpu.SemaphoreType.DMA,
            pltpu.SemaphoreType.DMA,
            pltpu.SemaphoreType.DMA,
        ],
    )
    def emb(xf_hbm, table_hbm, off_hbm, out_hbm, idx_v, rows_v, out_v, off_v,
            gsem0, gsem1, isem0, isem1, isem2, isem3, osem0, osem1):
        gsems = (gsem0, gsem1)
        isems = (isem0, isem1, isem2, isem3)
        osems = (osem0, osem1)
        wid = lax.axis_index("s") * nc + lax.axis_index("c")
        base_t = wid * tok_w

        pltpu.sync_copy(off_hbm, off_v)
        offv = off_v[...]

        def idx_desc(si, g):
            return pltpu.make_async_copy(
                xf_hbm.at[pl.ds((base_t + g * t_chunk) * q, rows_chunk)],
                idx_v.at[si], isems[si])

        def gather_descs(b, si):
            return [
                pltpu.make_async_copy(
                    table_hbm.at[idx_v.at[si, pl.ds(j * IDX_PER_DMA, IDX_PER_DMA)]],
                    rows_v.at[b, pl.ds(j * IDX_PER_DMA, IDX_PER_DMA)],
                    gsems[b],
                )
                for j in range(n_sub)
            ]

        def out_desc(b, g):
            return pltpu.make_async_copy(
                out_v.at[b], out_hbm.at[pl.ds(base_t + g * t_chunk, t_chunk)],
                osems[b])

        def stage(b, si, g):
            idx_desc(si, g).wait()

            @plsc.parallel_loop(0, rows_chunk, LANES, unroll=2)
            def off_body(s):
                idx_v[si, pl.ds(s, LANES)] = idx_v[si, pl.ds(s, LANES)] + offv

            for h in gather_descs(b, si):
                h.start()

        def compute(b, g):
            @pl.when(g >= NBUF)
            def _():
                out_desc(b, g).wait()

            @plsc.parallel_loop(0, t_chunk, 1, unroll=2)
            def tok_body(t):
                rbase = t * q
                for cc in range(f_v):
                    sl = pl.ds(cc * LANES, LANES)
                    vals = [rows_v[b, rbase + qq, sl] for qq in range(q)]
                    while len(vals) > 1:
                        vals = [vals[i] + vals[i + 1] for i in range(0, len(vals) - 1, 2)] + (
                            [vals[-1]] if len(vals) % 2 else [])
                    out_v[b, t, sl] = vals[0]

            out_desc(b, g).start()

        for g0 in range(NIDX):
            idx_desc(g0, g0).start()
        for g0 in range(NBUF):
            stage(g0 % NBUF, g0 % NIDX, g0)

        def loop_body(i, c):
            gg = i * NIDX
            for k in range(NIDX):
                g = gg + k
                b = k % NBUF
                si = k
                sj = (k + NBUF) % NIDX

                for h in gather_descs(b, si):
                    h.wait()

                @pl.when(g + NIDX < n_chunks)
                def _():
                    idx_desc(si, g + NIDX).start()

                compute(b, g)

                @pl.when(g + NBUF < n_chunks)
                def _():
                    stage(b, sj, g + NBUF)

            return c

        lax.fori_loop(0, n_chunks // NIDX, loop_body, 0)

        for g0 in range(NBUF):
            out_desc(g0 % NBUF, n_chunks - NBUF + g0).wait()

    return emb


def kernel(x, table, sum_over_quantizers):
    b, s, q = x.shape
    v, f = table.shape
    seg = v // q
    n_tok = b * s

    flag = jnp.asarray(sum_over_quantizers).astype(jnp.int32)
    off16 = (jnp.arange(LANES, dtype=jnp.int32) % q) * jnp.int32(seg) * flag

    xf = x.reshape(n_tok * q)
    out = _build(n_tok, q, f)(xf, table, off16)
    return out.reshape(b, s, f)


# speedup vs baseline: 1.0211x; 1.0195x over previous
import functools

import jax
import jax.numpy as jnp
from jax import lax
from jax.experimental import pallas as pl
from jax.experimental.pallas import tpu as pltpu
from jax.experimental.pallas import tpu_sc as plsc

LANES = 16
IDX_PER_DMA = 400
NBUF = 4
NIDX = 4


@functools.lru_cache(maxsize=None)
def _build(n_tok: int, q: int, f: int):
    info = plsc.get_sparse_core_info()
    nc, ns = info.num_cores, info.num_subcores
    nw = nc * ns

    assert f % LANES == 0
    assert n_tok % nw == 0
    tok_w = n_tok // nw
    t_chunk = 50
    while tok_w % (t_chunk * NIDX):
        t_chunk //= 2
    n_chunks = tok_w // t_chunk
    assert n_chunks >= NIDX
    rows_chunk = t_chunk * q
    assert rows_chunk % IDX_PER_DMA == 0
    n_sub = rows_chunk // IDX_PER_DMA
    f_v = f // LANES

    mesh = plsc.VectorSubcoreMesh(core_axis_name="c", subcore_axis_name="s")

    @functools.partial(
        pl.kernel,
        out_type=jax.ShapeDtypeStruct((n_tok, f), jnp.float32),
        mesh=mesh,
        compiler_params=pltpu.CompilerParams(use_tc_tiling_on_sc=False),
        scratch_types=[
            pltpu.VMEM((NIDX, rows_chunk), jnp.int32),
            pltpu.VMEM((NBUF, rows_chunk, f), jnp.float32),
            pltpu.VMEM((NBUF, t_chunk, f), jnp.float32),
            pltpu.VMEM((LANES,), jnp.int32),
            pltpu.SemaphoreType.DMA,
            pltpu.SemaphoreType.DMA,
            pltpu.SemaphoreType.DMA,
            pltpu.SemaphoreType.DMA,
            pltpu.SemaphoreType.DMA,
            pltpu.SemaphoreType.DMA,
            pltpu.SemaphoreType.DMA,
            pltpu.SemaphoreType.DMA,
            pltpu.SemaphoreType.DMA,
            pltpu.SemaphoreType.DMA,
            pltpu.SemaphoreType.DMA,
            pltpu.SemaphoreType.DMA,
        ],
    )
    def emb(xf_hbm, table_hbm, off_hbm, out_hbm, idx_v, rows_v, out_v, off_v,
            gsem0, gsem1, gsem2, gsem3, isem0, isem1, isem2, isem3,
            osem0, osem1, osem2, osem3):
        gsems = (gsem0, gsem1, gsem2, gsem3)
        isems = (isem0, isem1, isem2, isem3)
        osems = (osem0, osem1, osem2, osem3)
        wid = lax.axis_index("s") * nc + lax.axis_index("c")
        base_t = wid * tok_w

        pltpu.sync_copy(off_hbm, off_v)
        offv = off_v[...]

        def idx_desc(si, g):
            return pltpu.make_async_copy(
                xf_hbm.at[pl.ds((base_t + g * t_chunk) * q, rows_chunk)],
                idx_v.at[si], isems[si])

        def gather_descs(b, si):
            return [
                pltpu.make_async_copy(
                    table_hbm.at[idx_v.at[si, pl.ds(j * IDX_PER_DMA, IDX_PER_DMA)]],
                    rows_v.at[b, pl.ds(j * IDX_PER_DMA, IDX_PER_DMA)],
                    gsems[b],
                )
                for j in range(n_sub)
            ]

        def out_desc(b, g):
            return pltpu.make_async_copy(
                out_v.at[b], out_hbm.at[pl.ds(base_t + g * t_chunk, t_chunk)],
                osems[b])

        def stage(b, si, g):
            idx_desc(si, g).wait()

            @plsc.parallel_loop(0, rows_chunk, LANES, unroll=2)
            def off_body(s):
                idx_v[si, pl.ds(s, LANES)] = idx_v[si, pl.ds(s, LANES)] + offv

            for h in gather_descs(b, si):
                h.start()

        def compute(b, g):
            @pl.when(g >= NBUF)
            def _():
                out_desc(b, g).wait()

            @plsc.parallel_loop(0, t_chunk, 1, unroll=2)
            def tok_body(t):
                rbase = t * q
                for cc in range(f_v):
                    sl = pl.ds(cc * LANES, LANES)
                    vals = [rows_v[b, rbase + qq, sl] for qq in range(q)]
                    while len(vals) > 1:
                        vals = [vals[i] + vals[i + 1] for i in range(0, len(vals) - 1, 2)] + (
                            [vals[-1]] if len(vals) % 2 else [])
                    out_v[b, t, sl] = vals[0]

            out_desc(b, g).start()

        for g0 in range(NIDX):
            idx_desc(g0, g0).start()
        for g0 in range(NBUF):
            stage(g0 % NBUF, g0 % NIDX, g0)

        def loop_body(i, c):
            gg = i * NIDX
            for k in range(NIDX):
                g = gg + k
                b = k % NBUF
                si = k
                sj = (k + NBUF) % NIDX

                for h in gather_descs(b, si):
                    h.wait()

                @pl.when(g + NIDX < n_chunks)
                def _():
                    idx_desc(si, g + NIDX).start()

                compute(b, g)

                @pl.when(g + NBUF < n_chunks)
                def _():
                    stage(b, sj, g + NBUF)

            return c

        lax.fori_loop(0, n_chunks // NIDX, loop_body, 0)

        for g0 in range(NBUF):
            out_desc(g0 % NBUF, n_chunks - NBUF + g0).wait()

    return emb


def kernel(x, table, sum_over_quantizers):
    b, s, q = x.shape
    v, f = table.shape
    seg = v // q
    n_tok = b * s

    flag = jnp.asarray(sum_over_quantizers).astype(jnp.int32)
    off16 = (jnp.arange(LANES, dtype=jnp.int32) % q) * jnp.int32(seg) * flag

    xf = x.reshape(n_tok * q)
    out = _build(n_tok, q, f)(xf, table, off16)
    return out.reshape(b, s, f)
